# initial kernel scaffold (unmeasured)
import jax
import jax.numpy as jnp
from jax import lax
from jax.experimental import pallas as pl
from jax.experimental.pallas import tpu as pltpu


def kernel(
    x,
):
    def body(*refs):
        pass

    out_shape = jax.ShapeDtypeStruct(..., jnp.float32)
    return pl.pallas_call(body, out_shape=out_shape)(...)


# baseline (device time: 1557245 ns/iter reference)
import jax
import jax.numpy as jnp
from jax import lax
from jax.experimental import pallas as pl
from jax.experimental.pallas import tpu as pltpu

N_DEV = 32


def kernel(x):
    m_per, n = x.shape

    def body(x_ref, out_ref, copy_sem, send_sems, recv_sems):
        my = lax.axis_index("i")
        left = lax.rem(my + N_DEV - 1, N_DEV)
        right = lax.rem(my + 1, N_DEV)

        barrier_sem = pltpu.get_barrier_semaphore()
        pl.semaphore_signal(
            barrier_sem, inc=1, device_id=(left,),
            device_id_type=pl.DeviceIdType.MESH,
        )
        pl.semaphore_signal(
            barrier_sem, inc=1, device_id=(right,),
            device_id_type=pl.DeviceIdType.MESH,
        )
        pl.semaphore_wait(barrier_sem, 2)

        cp = pltpu.make_async_copy(
            x_ref, out_ref.at[pl.ds(my * m_per, m_per), :], copy_sem
        )
        cp.start()
        cp.wait()

        for h in range(N_DEV - 1):
            origin = lax.rem(my + N_DEV - h, N_DEV) if h else my
            rdma = pltpu.make_async_remote_copy(
                src_ref=out_ref.at[pl.ds(origin * m_per, m_per), :],
                dst_ref=out_ref.at[pl.ds(origin * m_per, m_per), :],
                send_sem=send_sems.at[h],
                recv_sem=recv_sems.at[h],
                device_id=(right,),
                device_id_type=pl.DeviceIdType.MESH,
            )
            rdma.start()
            rdma.wait()

    return pl.pallas_call(
        body,
        out_shape=jax.ShapeDtypeStruct((N_DEV * m_per, n), x.dtype),
        in_specs=[pl.BlockSpec(memory_space=pl.ANY)],
        out_specs=pl.BlockSpec(memory_space=pl.ANY),
        scratch_shapes=[
            pltpu.SemaphoreType.DMA,
            pltpu.SemaphoreType.DMA((N_DEV - 1,)),
            pltpu.SemaphoreType.DMA((N_DEV - 1,)),
        ],
        compiler_params=pltpu.CompilerParams(collective_id=0),
    )(x)


# device time: 849905 ns/iter; 1.8323x vs baseline; 1.8323x over previous
import jax
import jax.numpy as jnp
from jax import lax
from jax.experimental import pallas as pl
from jax.experimental.pallas import tpu as pltpu

N_DEV = 32
CW_HOPS = 16
CCW_HOPS = 15


def _build_cycle():
    order = []
    for z in range(4):
        for y in range(4):
            row = [(0, y, z), (1, y, z)] if y % 2 == 0 else [(1, y, z), (0, y, z)]
            order.extend(row)
    logical = {c: i for i, c in enumerate(order)}

    path = []
    for z in range(4):
        ys = range(4) if z % 2 == 0 else range(3, -1, -1)
        path.extend((y, z) for y in ys)
    cyc_coords = [(0, y, z) for (y, z) in path]
    cyc_coords += [(1, y, z) for (y, z) in reversed(path)]
    cycle = [logical[c] for c in cyc_coords]
    assert len(set(cycle)) == N_DEV
    return cycle


_CYCLE = _build_cycle()
_POS = [0] * N_DEV
for _p, _l in enumerate(_CYCLE):
    _POS[_l] = _p
_NEXT = [_CYCLE[(_POS[l] + 1) % N_DEV] for l in range(N_DEV)]
_PREV = [_CYCLE[(_POS[l] - 1) % N_DEV] for l in range(N_DEV)]


def _lut(table, idx):
    r = jnp.int32(table[0])
    for k in range(1, len(table)):
        r = lax.select(idx == k, jnp.int32(table[k]), r)
    return r


def kernel(x):
    m_per, n = x.shape

    def body(x_ref, out_ref, copy_sem, cw_send, cw_recv, ccw_send, ccw_recv):
        my = lax.axis_index("i")
        pos = _lut(_POS, my)
        nxt = _lut(_NEXT, my)
        prv = _lut(_PREV, my)

        barrier_sem = pltpu.get_barrier_semaphore()
        pl.semaphore_signal(
            barrier_sem, inc=1, device_id=(nxt,),
            device_id_type=pl.DeviceIdType.MESH,
        )
        pl.semaphore_signal(
            barrier_sem, inc=1, device_id=(prv,),
            device_id_type=pl.DeviceIdType.MESH,
        )
        pl.semaphore_wait(barrier_sem, 2)

        cp = pltpu.make_async_copy(
            x_ref, out_ref.at[pl.ds(my * m_per, m_per), :], copy_sem
        )
        cp.start()
        cp.wait()

        for h in range(CW_HOPS):
            o_cw = _lut(_CYCLE, lax.rem(pos - h + N_DEV, N_DEV))
            rdma_cw = pltpu.make_async_remote_copy(
                src_ref=out_ref.at[pl.ds(o_cw * m_per, m_per), :],
                dst_ref=out_ref.at[pl.ds(o_cw * m_per, m_per), :],
                send_sem=cw_send.at[h],
                recv_sem=cw_recv.at[h],
                device_id=(nxt,),
                device_id_type=pl.DeviceIdType.MESH,
            )
            rdma_cw.start()
            if h < CCW_HOPS:
                o_ccw = _lut(_CYCLE, lax.rem(pos + h, N_DEV))
                rdma_ccw = pltpu.make_async_remote_copy(
                    src_ref=out_ref.at[pl.ds(o_ccw * m_per, m_per), :],
                    dst_ref=out_ref.at[pl.ds(o_ccw * m_per, m_per), :],
                    send_sem=ccw_send.at[h],
                    recv_sem=ccw_recv.at[h],
                    device_id=(prv,),
                    device_id_type=pl.DeviceIdType.MESH,
                )
                rdma_ccw.start()
                rdma_ccw.wait()
            rdma_cw.wait()

    return pl.pallas_call(
        body,
        out_shape=jax.ShapeDtypeStruct((N_DEV * m_per, n), x.dtype),
        in_specs=[pl.BlockSpec(memory_space=pl.ANY)],
        out_specs=pl.BlockSpec(memory_space=pl.ANY),
        scratch_shapes=[
            pltpu.SemaphoreType.DMA,
            pltpu.SemaphoreType.DMA((CW_HOPS,)),
            pltpu.SemaphoreType.DMA((CW_HOPS,)),
            pltpu.SemaphoreType.DMA((CCW_HOPS,)),
            pltpu.SemaphoreType.DMA((CCW_HOPS,)),
        ],
        compiler_params=pltpu.CompilerParams(collective_id=0),
    )(x)


# device time: 824665 ns/iter; 1.8883x vs baseline; 1.0306x over previous
import jax
import jax.numpy as jnp
from jax import lax
from jax.experimental import pallas as pl
from jax.experimental.pallas import tpu as pltpu

N_DEV = 32
HOPS = 16


def _build_cycle():
    order = []
    for z in range(4):
        for y in range(4):
            row = [(0, y, z), (1, y, z)] if y % 2 == 0 else [(1, y, z), (0, y, z)]
            order.extend(row)
    logical = {c: i for i, c in enumerate(order)}

    path = []
    for z in range(4):
        ys = range(4) if z % 2 == 0 else range(3, -1, -1)
        path.extend((y, z) for y in ys)
    cyc_coords = [(0, y, z) for (y, z) in path]
    cyc_coords += [(1, y, z) for (y, z) in reversed(path)]
    cycle = [logical[c] for c in cyc_coords]
    assert len(set(cycle)) == N_DEV
    return cycle


_CYCLE = _build_cycle()
_POS = [0] * N_DEV
for _p, _l in enumerate(_CYCLE):
    _POS[_l] = _p
_NEXT = [_CYCLE[(_POS[l] + 1) % N_DEV] for l in range(N_DEV)]
_PREV = [_CYCLE[(_POS[l] - 1) % N_DEV] for l in range(N_DEV)]


def _lut(table, idx):
    r = jnp.int32(table[0])
    for k in range(1, len(table)):
        r = lax.select(idx == k, jnp.int32(table[k]), r)
    return r


def kernel(x):
    m_per, n = x.shape
    half = m_per // 2

    def body(x_ref, out_ref, copy_sem, cw_send, cw_recv, ccw_send, ccw_recv):
        my = lax.axis_index("i")
        pos = _lut(_POS, my)
        nxt = _lut(_NEXT, my)
        prv = _lut(_PREV, my)

        barrier_sem = pltpu.get_barrier_semaphore()
        pl.semaphore_signal(
            barrier_sem, inc=1, device_id=(nxt,),
            device_id_type=pl.DeviceIdType.MESH,
        )
        pl.semaphore_signal(
            barrier_sem, inc=1, device_id=(prv,),
            device_id_type=pl.DeviceIdType.MESH,
        )
        pl.semaphore_wait(barrier_sem, 2)

        def chunk(ref, origin, lo=0, rows=m_per):
            return ref.at[pl.ds(origin * m_per + lo, rows), :]

        def send(src, origin, sems, h, target, lo=0, rows=m_per):
            rdma = pltpu.make_async_remote_copy(
                src_ref=chunk(src, origin, lo, rows)
                if src is out_ref
                else src.at[pl.ds(lo, rows), :],
                dst_ref=chunk(out_ref, origin, lo, rows),
                send_sem=sems[0].at[h],
                recv_sem=sems[1].at[h],
                device_id=(target,),
                device_id_type=pl.DeviceIdType.MESH,
            )
            rdma.start()
            return rdma

        sends = []
        sends.append(send(x_ref, my, (cw_send, cw_recv), 0, nxt))
        sends.append(send(x_ref, my, (ccw_send, ccw_recv), 0, prv))
        cp = pltpu.make_async_copy(
            x_ref, out_ref.at[pl.ds(my * m_per, m_per), :], copy_sem
        )
        cp.start()

        cw_waits, ccw_waits = [sends[0]], [sends[1]]
        for h in range(1, HOPS):
            o_cw = _lut(_CYCLE, lax.rem(pos - h + N_DEV, N_DEV))
            o_ccw = _lut(_CYCLE, lax.rem(pos + h, N_DEV))
            cw_waits[h - 1].wait_recv()
            sends.append(
                send(out_ref, o_cw, (cw_send, cw_recv), h, nxt,
                     lo=0, rows=half if h == HOPS - 1 else m_per)
            )
            cw_waits.append(sends[-1])
            ccw_waits[h - 1].wait_recv()
            sends.append(
                send(out_ref, o_ccw, (ccw_send, ccw_recv), h, prv,
                     lo=half if h == HOPS - 1 else 0,
                     rows=half if h == HOPS - 1 else m_per)
            )
            ccw_waits.append(sends[-1])

        cw_waits[-1].wait_recv()
        ccw_waits[-1].wait_recv()
        for r in sends:
            r.wait_send()
        cp.wait()

    return pl.pallas_call(
        body,
        out_shape=jax.ShapeDtypeStruct((N_DEV * m_per, n), x.dtype),
        in_specs=[pl.BlockSpec(memory_space=pl.ANY)],
        out_specs=pl.BlockSpec(memory_space=pl.ANY),
        scratch_shapes=[
            pltpu.SemaphoreType.DMA,
            pltpu.SemaphoreType.DMA((HOPS,)),
            pltpu.SemaphoreType.DMA((HOPS,)),
            pltpu.SemaphoreType.DMA((HOPS,)),
            pltpu.SemaphoreType.DMA((HOPS,)),
        ],
        compiler_params=pltpu.CompilerParams(collective_id=0),
    )(x)


# device time: 794600 ns/iter; 1.9598x vs baseline; 1.0378x over previous
import jax
import jax.numpy as jnp
from jax import lax
from jax.experimental import pallas as pl
from jax.experimental.pallas import tpu as pltpu

N_DEV = 32
HOPS = 16


def _build_cycle():
    order = []
    for z in range(4):
        for y in range(4):
            row = [(0, y, z), (1, y, z)] if y % 2 == 0 else [(1, y, z), (0, y, z)]
            order.extend(row)
    logical = {c: i for i, c in enumerate(order)}

    path = []
    for z in range(4):
        ys = range(4) if z % 2 == 0 else range(3, -1, -1)
        path.extend((y, z) for y in ys)
    cyc_coords = [(0, y, z) for (y, z) in path]
    cyc_coords += [(1, y, z) for (y, z) in reversed(path)]
    cycle = [logical[c] for c in cyc_coords]
    assert len(set(cycle)) == N_DEV
    return cycle


_CYCLE = _build_cycle()
_POS = [0] * N_DEV
for _p, _l in enumerate(_CYCLE):
    _POS[_l] = _p
_NEXT = [_CYCLE[(_POS[l] + 1) % N_DEV] for l in range(N_DEV)]
_PREV = [_CYCLE[(_POS[l] - 1) % N_DEV] for l in range(N_DEV)]


def _lut(table, idx):
    r = jnp.int32(table[0])
    for k in range(1, len(table)):
        r = lax.select(idx == k, jnp.int32(table[k]), r)
    return r


def kernel(x):
    m_per, n = x.shape
    half = m_per // 2

    def body(x_ref, out_ref, copy_sem, cwt_s, cwt_r, cwb_s, cwb_r,
             cct_s, cct_r, ccb_s, ccb_r):
        my = lax.axis_index("i")
        pos = _lut(_POS, my)
        nxt = _lut(_NEXT, my)
        prv = _lut(_PREV, my)

        barrier_sem = pltpu.get_barrier_semaphore()
        pl.semaphore_signal(
            barrier_sem, inc=1, device_id=(nxt,),
            device_id_type=pl.DeviceIdType.MESH,
        )
        pl.semaphore_signal(
            barrier_sem, inc=1, device_id=(prv,),
            device_id_type=pl.DeviceIdType.MESH,
        )
        pl.semaphore_wait(barrier_sem, 2)

        o_cw = [my] + [
            _lut(_CYCLE, lax.rem(pos - h + N_DEV, N_DEV)) for h in range(1, HOPS)
        ]
        o_cc = [my] + [
            _lut(_CYCLE, lax.rem(pos + h, N_DEV)) for h in range(1, HOPS)
        ]

        def send(src_ref, src_lo, origin, lo, sems, h, target):
            rdma = pltpu.make_async_remote_copy(
                src_ref=src_ref.at[pl.ds(src_lo, half), :],
                dst_ref=out_ref.at[pl.ds(origin * m_per + lo, half), :],
                send_sem=sems[0].at[h],
                recv_sem=sems[1].at[h],
                device_id=(target,),
                device_id_type=pl.DeviceIdType.MESH,
            )
            rdma.start()
            return rdma

        chains = {
            "cwt": dict(sems=(cwt_s, cwt_r), tgt=nxt, org=o_cw, lo=0, hops=HOPS),
            "cct": dict(sems=(cct_s, cct_r), tgt=prv, org=o_cc, lo=0, hops=HOPS - 1),
            "cwb": dict(sems=(cwb_s, cwb_r), tgt=nxt, org=o_cw, lo=half, hops=HOPS - 1),
            "ccb": dict(sems=(ccb_s, ccb_r), tgt=prv, org=o_cc, lo=half, hops=HOPS),
        }

        sends = []
        last = {}
        for k, c in chains.items():
            last[k] = send(x_ref, c["lo"], my, c["lo"], c["sems"], 0, c["tgt"])
            sends.append(last[k])
        cp = pltpu.make_async_copy(
            x_ref, out_ref.at[pl.ds(my * m_per, m_per), :], copy_sem
        )
        cp.start()

        for h in range(1, HOPS):
            for k, c in chains.items():
                if h >= c["hops"]:
                    continue
                last[k].wait_recv()
                o = c["org"][h]
                last[k] = send(
                    out_ref, o * m_per + c["lo"], o, c["lo"], c["sems"], h, c["tgt"]
                )
                sends.append(last[k])

        for k in chains:
            last[k].wait_recv()
        for r in sends:
            r.wait_send()
        cp.wait()

    return pl.pallas_call(
        body,
        out_shape=jax.ShapeDtypeStruct((N_DEV * m_per, n), x.dtype),
        in_specs=[pl.BlockSpec(memory_space=pl.ANY)],
        out_specs=pl.BlockSpec(memory_space=pl.ANY),
        scratch_shapes=[pltpu.SemaphoreType.DMA]
        + [pltpu.SemaphoreType.DMA((HOPS,)) for _ in range(8)],
        compiler_params=pltpu.CompilerParams(collective_id=0),
    )(x)


# device time: 793137 ns/iter; 1.9634x vs baseline; 1.0018x over previous
import jax
import jax.numpy as jnp
from jax import lax
from jax.experimental import pallas as pl
from jax.experimental.pallas import tpu as pltpu

N_DEV = 32
HOPS = 16
S = 4


def _build_cycle():
    order = []
    for z in range(4):
        for y in range(4):
            row = [(0, y, z), (1, y, z)] if y % 2 == 0 else [(1, y, z), (0, y, z)]
            order.extend(row)
    logical = {c: i for i, c in enumerate(order)}

    path = []
    for z in range(4):
        ys = range(4) if z % 2 == 0 else range(3, -1, -1)
        path.extend((y, z) for y in ys)
    cyc_coords = [(0, y, z) for (y, z) in path]
    cyc_coords += [(1, y, z) for (y, z) in reversed(path)]
    cycle = [logical[c] for c in cyc_coords]
    assert len(set(cycle)) == N_DEV
    return cycle


_CYCLE = _build_cycle()
_POS = [0] * N_DEV
for _p, _l in enumerate(_CYCLE):
    _POS[_l] = _p
_NEXT = [_CYCLE[(_POS[l] + 1) % N_DEV] for l in range(N_DEV)]
_PREV = [_CYCLE[(_POS[l] - 1) % N_DEV] for l in range(N_DEV)]


def _lut(table, idx):
    r = jnp.int32(table[0])
    for k in range(1, len(table)):
        r = lax.select(idx == k, jnp.int32(table[k]), r)
    return r


def kernel(x):
    m_per, n = x.shape
    sub = m_per // S

    def body(x_ref, out_ref, copy_sem, *sems):
        my = lax.axis_index("i")
        pos = _lut(_POS, my)
        nxt = _lut(_NEXT, my)
        prv = _lut(_PREV, my)

        barrier_sem = pltpu.get_barrier_semaphore()
        pl.semaphore_signal(
            barrier_sem, inc=1, device_id=(nxt,),
            device_id_type=pl.DeviceIdType.MESH,
        )
        pl.semaphore_signal(
            barrier_sem, inc=1, device_id=(prv,),
            device_id_type=pl.DeviceIdType.MESH,
        )
        pl.semaphore_wait(barrier_sem, 2)

        o_cw = [my] + [
            _lut(_CYCLE, lax.rem(pos - h + N_DEV, N_DEV)) for h in range(1, HOPS)
        ]
        o_cc = [my] + [
            _lut(_CYCLE, lax.rem(pos + h, N_DEV)) for h in range(1, HOPS)
        ]

        def send(src_ref, src_lo, dst_lo, sems2, h, target):
            rdma = pltpu.make_async_remote_copy(
                src_ref=src_ref.at[pl.ds(src_lo, sub), :],
                dst_ref=out_ref.at[pl.ds(dst_lo, sub), :],
                send_sem=sems2[0].at[h],
                recv_sem=sems2[1].at[h],
                device_id=(target,),
                device_id_type=pl.DeviceIdType.MESH,
            )
            rdma.start()
            return rdma

        chains = []
        for q in range(S):
            chains.append(dict(
                sems=(sems[4 * q], sems[4 * q + 1]), tgt=nxt, org=o_cw,
                lo=q * sub, hops=HOPS if q < S // 2 else HOPS - 1,
            ))
            chains.append(dict(
                sems=(sems[4 * q + 2], sems[4 * q + 3]), tgt=prv, org=o_cc,
                lo=q * sub, hops=HOPS if q >= S // 2 else HOPS - 1,
            ))

        sends = []
        for c in chains:
            c["last"] = send(x_ref, c["lo"], my * m_per + c["lo"], c["sems"], 0, c["tgt"])
            sends.append(c["last"])
        cp = pltpu.make_async_copy(
            x_ref, out_ref.at[pl.ds(my * m_per, m_per), :], copy_sem
        )
        cp.start()

        for h in range(1, HOPS):
            for c in chains:
                if h >= c["hops"]:
                    continue
                c["last"].wait_recv()
                o = c["org"][h]
                c["last"] = send(
                    out_ref, o * m_per + c["lo"], o * m_per + c["lo"],
                    c["sems"], h, c["tgt"],
                )
                sends.append(c["last"])

        for c in chains:
            c["last"].wait_recv()
        for r in sends:
            r.wait_send()
        cp.wait()

    return pl.pallas_call(
        body,
        out_shape=jax.ShapeDtypeStruct((N_DEV * m_per, n), x.dtype),
        in_specs=[pl.BlockSpec(memory_space=pltpu.MemorySpace.VMEM)],
        out_specs=pl.BlockSpec(memory_space=pl.ANY),
        scratch_shapes=[pltpu.SemaphoreType.DMA]
        + [pltpu.SemaphoreType.DMA((HOPS,)) for _ in range(4 * S)],
        compiler_params=pltpu.CompilerParams(collective_id=0),
    )(x)


# device time: 732443 ns/iter; 2.1261x vs baseline; 1.0829x over previous
import jax
import jax.numpy as jnp
from jax import lax
from jax.experimental import pallas as pl
from jax.experimental.pallas import tpu as pltpu

N_DEV = 32
NZ = 4
NR = 8
R_HOPS = 4

RINGP = [0, 1, 2, 5, 6, 7, 4, 3]
RPOS = [0] * NR
for _r, _p in enumerate(RINGP):
    RPOS[_p] = _r


def _lut(table, idx):
    r = jnp.int32(table[0])
    for k in range(1, len(table)):
        r = lax.select(idx == k, jnp.int32(table[k]), r)
    return r


def kernel(x):
    m_per, n = x.shape

    def body(x_ref, out_ref, copy_sem, zu_s, zu_r, zd_s, zd_r,
             cw_s, cw_r, cc_s, cc_r):
        my = lax.axis_index("i")
        my_z = my // NR
        my_p = lax.rem(my, NR)
        my_r = _lut(RPOS, my_p)
        nxt = my_z * NR + _lut(RINGP, lax.rem(my_r + 1, NR))
        prv = my_z * NR + _lut(RINGP, lax.rem(my_r + NR - 1, NR))
        zup = my + NR
        zdn = my - NR
        has_up = my_z < NZ - 1
        has_dn = my_z > 0

        barrier_sem = pltpu.get_barrier_semaphore()
        for tgt in (nxt, prv):
            pl.semaphore_signal(
                barrier_sem, inc=1, device_id=(tgt,),
                device_id_type=pl.DeviceIdType.MESH,
            )
        for cond, tgt in ((has_up, zup), (has_dn, zdn)):
            @pl.when(cond)
            def _(tgt=tgt):
                pl.semaphore_signal(
                    barrier_sem, inc=1, device_id=(tgt,),
                    device_id_type=pl.DeviceIdType.MESH,
                )
            @pl.when(jnp.logical_not(cond))
            def _():
                pl.semaphore_signal(barrier_sem, inc=1)
        pl.semaphore_wait(barrier_sem, 4)

        def mk(src_ref, src_lo, dst_lo, ssem, rsem, target):
            return pltpu.make_async_remote_copy(
                src_ref=src_ref.at[pl.ds(src_lo, m_per), :],
                dst_ref=out_ref.at[pl.ds(dst_lo, m_per), :],
                send_sem=ssem,
                recv_sem=rsem,
                device_id=(target,),
                device_id_type=pl.DeviceIdType.MESH,
            )

        def rwait(dst_lo, rsem):
            mk(out_ref, dst_lo, dst_lo, rsem, rsem, my).wait_recv()

        sends = []

        def start(desc, cond=None):
            if cond is None:
                desc.start()
            else:
                @pl.when(cond)
                def _():
                    desc.start()
            sends.append((desc, cond))

        pcw = [_lut(RINGP, lax.rem(my_r - h + NR, NR)) for h in range(R_HOPS + 1)]
        pcc = [_lut(RINGP, lax.rem(my_r + h, NR)) for h in range(R_HOPS + 1)]

        def ring_send(q, h, to_nxt):
            plane = pcw[h] if to_nxt else pcc[h]
            ol = q * NR + plane
            s, r = (cw_s, cw_r) if to_nxt else (cc_s, cc_r)
            src = x_ref if h == 0 and q is my_z else out_ref
            desc = mk(
                x_ref if src is x_ref else out_ref,
                0 if src is x_ref else ol * m_per,
                ol * m_per,
                s.at[q * R_HOPS + h],
                r.at[q * R_HOPS + h],
                nxt if to_nxt else prv,
            )
            return desc

        start(ring_send(my_z, 0, True))
        start(ring_send(my_z, 0, False))
        start(mk(x_ref, 0, my * m_per, zu_s.at[my_z], zu_r.at[my_z], zup),
              has_up)
        start(mk(x_ref, 0, my * m_per, zd_s.at[my_z], zd_r.at[my_z], zdn),
              has_dn)
        cp = pltpu.make_async_copy(
            x_ref, out_ref.at[pl.ds(my * m_per, m_per), :], copy_sem
        )
        cp.start()

        for d in range(1, NZ):
            for frm_below in (True, False):
                o = my_z - d if frm_below else my_z + d
                cond = (o >= 0) if frm_below else (o <= NZ - 1)
                col = o * NR + my_p
                rsem = zu_r if frm_below else zd_r

                @pl.when(cond)
                def _(o=o, col=col, rsem=rsem):
                    rwait(col * m_per, rsem.at[o])

                fwd_ok = jnp.logical_and(cond, has_up if frm_below else has_dn)
                fsem_s = zu_s if frm_below else zd_s
                fsem_r = zu_r if frm_below else zd_r
                start(
                    mk(out_ref, col * m_per, col * m_per,
                       fsem_s.at[o], fsem_r.at[o],
                       zup if frm_below else zdn),
                    fwd_ok,
                )
                start(ring_send(o, 0, True), cond)
                start(ring_send(o, 0, False), cond)

        def last_hop(q, to_nxt):
            if to_nxt:
                return R_HOPS - 1 if q < 2 else R_HOPS - 2
            return R_HOPS - 1 if q >= 2 else R_HOPS - 2

        for h in range(1, R_HOPS):
            for q in range(4):
                for to_nxt in (True, False):
                    if h > last_hop(q, to_nxt):
                        continue
                    r = cw_r if to_nxt else cc_r
                    plane = pcw[h] if to_nxt else pcc[h]
                    rwait((q * NR + plane) * m_per, r.at[q * R_HOPS + h - 1])
                    start(ring_send(q, h, to_nxt))

        for q in range(4):
            for to_nxt in (True, False):
                lh = last_hop(q, to_nxt)
                r = cw_r if to_nxt else cc_r
                plane = pcw[lh + 1] if to_nxt else pcc[lh + 1]
                rwait((q * NR + plane) * m_per, r.at[q * R_HOPS + lh])

        for desc, cond in sends:
            if cond is None:
                desc.wait_send()
            else:
                @pl.when(cond)
                def _(desc=desc):
                    desc.wait_send()
        cp.wait()

    return pl.pallas_call(
        body,
        out_shape=jax.ShapeDtypeStruct((N_DEV * m_per, n), x.dtype),
        in_specs=[pl.BlockSpec(memory_space=pltpu.MemorySpace.VMEM)],
        out_specs=pl.BlockSpec(memory_space=pl.ANY),
        scratch_shapes=[pltpu.SemaphoreType.DMA]
        + [pltpu.SemaphoreType.DMA((NZ,)) for _ in range(4)]
        + [pltpu.SemaphoreType.DMA((NZ * R_HOPS,)) for _ in range(4)],
        compiler_params=pltpu.CompilerParams(collective_id=0),
    )(x)
